# Initial kernel scaffold; baseline (speedup 1.0000x reference)
import jax
import jax.numpy as jnp
from jax.experimental import pallas as pl


def kernel(rpn_probs, rpn_bbox, anchors):
    raise NotImplementedError("write your pallas kernel here")


# speedup vs baseline: 19.4020x; 19.4020x over previous
import functools

import jax
import jax.numpy as jnp
from jax import lax
from jax.experimental import pallas as pl
from jax.experimental.pallas import tpu as pltpu
from jax.experimental.pallas import tpu_sc as plsc

N_IMG = 4
N_PRE = 6000
N_OUT = 1000
KPAD = 1008
THR = 0.7
L = 16


def _vgather(vec, idx):
    return lax.gather(
        vec,
        idx[:, None],
        lax.GatherDimensionNumbers(
            offset_dims=(), collapsed_slice_dims=(0,), start_index_map=(0,)
        ),
        (1,),
        mode=lax.GatherScatterMode.PROMISE_IN_BOUNDS,
    )


def _nms_body(a_hbm, out_hbm, a_v, bx_v, kept_v, outv, n_ref):
    w = lax.axis_index("s") * 2 + lax.axis_index("c")
    iota = lax.iota(jnp.int32, L)
    zero16 = jnp.zeros((L,), jnp.float32)

    @pl.when(w < N_IMG)
    def _():
        pltpu.sync_copy(a_hbm.at[w], a_v)

        def dec(j, carry):
            sl = pl.ds(pl.multiple_of(j * L, L), L)
            ay1 = a_v[0, sl]
            ax1 = a_v[1, sl]
            ay2 = a_v[2, sl]
            ax2 = a_v[3, sl]
            dy = a_v[4, sl] * 0.1
            dx = a_v[5, sl] * 0.1
            dh = a_v[6, sl] * 0.2
            dw = a_v[7, sl] * 0.2
            h = ay2 - ay1
            wd = ax2 - ax1
            cy = ay1 + 0.5 * h + dy * h
            cx = ax1 + 0.5 * wd + dx * wd
            h2 = h * jnp.exp(dh)
            w2 = wd * jnp.exp(dw)
            vy1 = cy - 0.5 * h2
            vx1 = cx - 0.5 * w2
            vy2 = vy1 + h2
            vx2 = vx1 + w2
            vy1 = jnp.minimum(jnp.maximum(vy1, 0.0), 1.0)
            vx1 = jnp.minimum(jnp.maximum(vx1, 0.0), 1.0)
            vy2 = jnp.minimum(jnp.maximum(vy2, 0.0), 1.0)
            vx2 = jnp.minimum(jnp.maximum(vx2, 0.0), 1.0)
            bx_v[0, sl] = vy1
            bx_v[1, sl] = vx1
            bx_v[2, sl] = vy2
            bx_v[3, sl] = vx2
            bx_v[4, sl] = (vy2 - vy1) * (vx2 - vx1)
            return carry

        lax.fori_loop(0, N_PRE // L, dec, 0)

        def zout(j, carry):
            outv[pl.ds(pl.multiple_of(j * L, L), L)] = zero16
            return carry

        lax.fori_loop(0, (N_OUT * 4) // L, zout, 0)

        def zkept(j, carry):
            sl = pl.ds(pl.multiple_of(j * L, L), L)
            kept_v[0, sl] = zero16
            kept_v[1, sl] = zero16
            kept_v[2, sl] = zero16
            kept_v[3, sl] = zero16
            kept_v[4, sl] = zero16
            return carry

        lax.fori_loop(0, KPAD // L, zkept, 0)
        n_ref[0] = jnp.int32(0)

        def cand(i, carry):
            @pl.when(n_ref[0] < N_OUT)
            def _():
                base = pl.ds(pl.multiple_of((i >> 4) * L, L), L)
                lanev = (i & (L - 1)) + 0 * iota
                by1 = _vgather(bx_v[0, base], lanev)
                bx1 = _vgather(bx_v[1, base], lanev)
                by2 = _vgather(bx_v[2, base], lanev)
                bx2 = _vgather(bx_v[3, base], lanev)
                ba = _vgather(bx_v[4, base], lanev)
                n = n_ref[0]
                nch = (n + (L - 1)) >> 4

                def ibody(c, f):
                    ksl = pl.ds(pl.multiple_of(c * L, L), L)
                    iy1 = jnp.maximum(by1, kept_v[0, ksl])
                    ix1 = jnp.maximum(bx1, kept_v[1, ksl])
                    iy2 = jnp.minimum(by2, kept_v[2, ksl])
                    ix2 = jnp.minimum(bx2, kept_v[3, ksl])
                    inter = jnp.maximum(iy2 - iy1, 0.0) * jnp.maximum(
                        ix2 - ix1, 0.0
                    )
                    iou = inter / (ba + kept_v[4, ksl] - inter + 1e-8)
                    return jnp.maximum(f, iou)

                miou = lax.fori_loop(0, nch, ibody, zero16)
                for sh in (8, 4, 2, 1):
                    miou = jnp.maximum(miou, _vgather(miou, iota ^ sh))
                keep = miou[0] <= THR

                rsel = jnp.where(keep, n & (L - 1), L)
                ksl = pl.ds(pl.multiple_of((n >> 4) * L, L), L)
                m = iota == rsel
                kept_v[0, ksl] = jnp.where(m, by1, kept_v[0, ksl])
                kept_v[1, ksl] = jnp.where(m, bx1, kept_v[1, ksl])
                kept_v[2, ksl] = jnp.where(m, by2, kept_v[2, ksl])
                kept_v[3, ksl] = jnp.where(m, bx2, kept_v[3, ksl])
                kept_v[4, ksl] = jnp.where(m, ba, kept_v[4, ksl])

                n4 = n * 4
                rb = jnp.where(keep, n4 & (L - 1), 2 * L)
                osl = pl.ds(pl.multiple_of((n4 >> 4) * L, L), L)
                ov = outv[osl]
                ov = jnp.where(iota == rb, by1, ov)
                ov = jnp.where(iota == rb + 1, bx1, ov)
                ov = jnp.where(iota == rb + 2, by2, ov)
                ov = jnp.where(iota == rb + 3, bx2, ov)
                outv[osl] = ov
                n_ref[0] = jnp.where(keep, n + 1, n)

            return carry

        lax.fori_loop(0, N_PRE, cand, 0)
        pltpu.sync_copy(outv, out_hbm.at[w])


def kernel(rpn_probs, rpn_bbox, anchors):
    scores = rpn_probs[:, :, 1]
    _, ix = lax.top_k(scores, N_PRE)
    anchors_t = jnp.take_along_axis(anchors, ix[..., None], axis=1)
    deltas_t = jnp.take_along_axis(rpn_bbox, ix[..., None], axis=1)
    a = jnp.concatenate(
        [anchors_t.transpose(0, 2, 1), deltas_t.transpose(0, 2, 1)], axis=1
    )

    mesh = plsc.VectorSubcoreMesh(core_axis_name="c", subcore_axis_name="s")
    k = functools.partial(
        pl.kernel,
        out_type=jax.ShapeDtypeStruct((N_IMG, N_OUT * 4), jnp.float32),
        mesh=mesh,
        scratch_types=[
            pltpu.VMEM((8, N_PRE), jnp.float32),
            pltpu.VMEM((5, N_PRE), jnp.float32),
            pltpu.VMEM((5, KPAD), jnp.float32),
            pltpu.VMEM((N_OUT * 4,), jnp.float32),
            pltpu.SMEM((1,), jnp.int32),
        ],
    )(_nms_body)
    out = k(a)
    return out.reshape(N_IMG, N_OUT, 4)


# speedup vs baseline: 24.2886x; 1.2519x over previous
import functools

import jax
import jax.numpy as jnp
from jax import lax
from jax.experimental import pallas as pl
from jax.experimental.pallas import tpu as pltpu
from jax.experimental.pallas import tpu_sc as plsc

N_IMG = 4
N_ANC = 20000
N_PRE = 6000
N_OUT = 1000
KPAD = 1024
THR = 0.7
L = 16


def _vgather(vec, idx):
    return lax.gather(
        vec,
        idx[:, None],
        lax.GatherDimensionNumbers(
            offset_dims=(), collapsed_slice_dims=(0,), start_index_map=(0,)
        ),
        (1,),
        mode=lax.GatherScatterMode.PROMISE_IN_BOUNDS,
    )


def _body(tabs_hbm, ix_hbm, out_hbm, tab_v, idx_v, gath_v, a_v,
          kept_v, outv, n_ref, shared):
    c = lax.axis_index("c")
    s = lax.axis_index("s")
    img_local = s // 8
    img = 2 * c + img_local
    coord = s % 8
    iota = lax.iota(jnp.int32, L)
    zero16 = jnp.zeros((L,), jnp.float32)

    pltpu.sync_copy(tabs_hbm.at[img, coord], tab_v)
    pltpu.sync_copy(ix_hbm.at[img], idx_v)

    def gath(j, carry):
        sl = pl.ds(pl.multiple_of(j * L, L), L)
        ivec = idx_v[sl]
        acc = zero16
        for t in range(L):
            ixt = ivec[t]
            ch = tab_v[pl.ds(pl.multiple_of((ixt >> 4) * L, L), L)]
            val = _vgather(ch, (ixt & (L - 1)) + 0 * iota)
            acc = jnp.where(iota == t, val, acc)
        gath_v[sl] = acc
        return carry

    lax.fori_loop(0, N_PRE // L, gath, 0)
    pltpu.sync_copy(gath_v, shared.at[img_local, coord])
    plsc.subcore_barrier()

    @pl.when(coord == 0)
    def _():
        pltpu.sync_copy(shared.at[img_local], a_v)

        def dec(j, carry):
            sl = pl.ds(pl.multiple_of(j * L, L), L)
            ay1 = a_v[0, sl]
            ax1 = a_v[1, sl]
            ay2 = a_v[2, sl]
            ax2 = a_v[3, sl]
            dy = a_v[4, sl] * 0.1
            dx = a_v[5, sl] * 0.1
            dh = a_v[6, sl] * 0.2
            dw = a_v[7, sl] * 0.2
            h = ay2 - ay1
            wd = ax2 - ax1
            cy = ay1 + 0.5 * h + dy * h
            cx = ax1 + 0.5 * wd + dx * wd
            h2 = h * jnp.exp(dh)
            w2 = wd * jnp.exp(dw)
            vy1 = cy - 0.5 * h2
            vx1 = cx - 0.5 * w2
            vy2 = vy1 + h2
            vx2 = vx1 + w2
            vy1 = jnp.minimum(jnp.maximum(vy1, 0.0), 1.0)
            vx1 = jnp.minimum(jnp.maximum(vx1, 0.0), 1.0)
            vy2 = jnp.minimum(jnp.maximum(vy2, 0.0), 1.0)
            vx2 = jnp.minimum(jnp.maximum(vx2, 0.0), 1.0)
            a_v[0, sl] = vy1
            a_v[1, sl] = vx1
            a_v[2, sl] = vy2
            a_v[3, sl] = vx2
            a_v[4, sl] = (vy2 - vy1) * (vx2 - vx1)
            return carry

        lax.fori_loop(0, N_PRE // L, dec, 0)

        def zout(j, carry):
            outv[pl.ds(pl.multiple_of(j * L, L), L)] = zero16
            return carry

        lax.fori_loop(0, (N_OUT * 4) // L, zout, 0)

        def zkept(j, carry):
            sl = pl.ds(pl.multiple_of(j * L, L), L)
            kept_v[0, sl] = zero16
            kept_v[1, sl] = zero16
            kept_v[2, sl] = zero16
            kept_v[3, sl] = zero16
            kept_v[4, sl] = zero16
            return carry

        lax.fori_loop(0, KPAD // L, zkept, 0)
        n_ref[0] = jnp.int32(0)

        def cand(i, carry):
            @pl.when(n_ref[0] < N_OUT)
            def _():
                base = pl.ds(pl.multiple_of((i >> 4) * L, L), L)
                lanev = (i & (L - 1)) + 0 * iota
                by1 = _vgather(a_v[0, base], lanev)
                bx1 = _vgather(a_v[1, base], lanev)
                by2 = _vgather(a_v[2, base], lanev)
                bx2 = _vgather(a_v[3, base], lanev)
                ba = _vgather(a_v[4, base], lanev)
                n = n_ref[0]
                nch = (n + (2 * L - 1)) >> 5

                def ibody(cc, fs):
                    f1, f2 = fs
                    base0 = pl.multiple_of(cc * 2 * L, L)
                    base1 = pl.multiple_of(cc * 2 * L + L, L)
                    k0 = pl.ds(base0, L)
                    k1 = pl.ds(base1, L)
                    a_iy1 = jnp.maximum(by1, kept_v[0, k0])
                    b_iy1 = jnp.maximum(by1, kept_v[0, k1])
                    a_ix1 = jnp.maximum(bx1, kept_v[1, k0])
                    b_ix1 = jnp.maximum(bx1, kept_v[1, k1])
                    a_iy2 = jnp.minimum(by2, kept_v[2, k0])
                    b_iy2 = jnp.minimum(by2, kept_v[2, k1])
                    a_ix2 = jnp.minimum(bx2, kept_v[3, k0])
                    b_ix2 = jnp.minimum(bx2, kept_v[3, k1])
                    a_in = jnp.maximum(a_iy2 - a_iy1, 0.0) * jnp.maximum(
                        a_ix2 - a_ix1, 0.0
                    )
                    b_in = jnp.maximum(b_iy2 - b_iy1, 0.0) * jnp.maximum(
                        b_ix2 - b_ix1, 0.0
                    )
                    a_iou = a_in / (ba + kept_v[4, k0] - a_in + 1e-8)
                    b_iou = b_in / (ba + kept_v[4, k1] - b_in + 1e-8)
                    return jnp.maximum(f1, a_iou), jnp.maximum(f2, b_iou)

                m1, m2 = lax.fori_loop(0, nch, ibody, (zero16, zero16))
                miou = jnp.maximum(m1, m2)
                for sh in (8, 4, 2, 1):
                    miou = jnp.maximum(miou, _vgather(miou, iota ^ sh))
                keep = miou[0] <= THR

                rsel = jnp.where(keep, n & (L - 1), L)
                ksl = pl.ds(pl.multiple_of((n >> 4) * L, L), L)
                m = iota == rsel
                kept_v[0, ksl] = jnp.where(m, by1, kept_v[0, ksl])
                kept_v[1, ksl] = jnp.where(m, bx1, kept_v[1, ksl])
                kept_v[2, ksl] = jnp.where(m, by2, kept_v[2, ksl])
                kept_v[3, ksl] = jnp.where(m, bx2, kept_v[3, ksl])
                kept_v[4, ksl] = jnp.where(m, ba, kept_v[4, ksl])

                n4 = n * 4
                rb = jnp.where(keep, n4 & (L - 1), 2 * L)
                osl = pl.ds(pl.multiple_of((n4 >> 4) * L, L), L)
                ov = outv[osl]
                ov = jnp.where(iota == rb, by1, ov)
                ov = jnp.where(iota == rb + 1, bx1, ov)
                ov = jnp.where(iota == rb + 2, by2, ov)
                ov = jnp.where(iota == rb + 3, bx2, ov)
                outv[osl] = ov
                n_ref[0] = jnp.where(keep, n + 1, n)

            return carry

        lax.fori_loop(0, N_PRE, cand, 0)
        pltpu.sync_copy(outv, out_hbm.at[img])


def kernel(rpn_probs, rpn_bbox, anchors):
    scores = rpn_probs[:, :, 1]
    _, ix = lax.top_k(scores, N_PRE)
    tabs = jnp.concatenate(
        [anchors.transpose(0, 2, 1), rpn_bbox.transpose(0, 2, 1)], axis=1
    )

    mesh = plsc.VectorSubcoreMesh(core_axis_name="c", subcore_axis_name="s")
    k = functools.partial(
        pl.kernel,
        out_type=jax.ShapeDtypeStruct((N_IMG, N_OUT * 4), jnp.float32),
        mesh=mesh,
        scratch_types=[
            pltpu.VMEM((N_ANC,), jnp.float32),
            pltpu.VMEM((N_PRE,), jnp.int32),
            pltpu.VMEM((N_PRE,), jnp.float32),
            pltpu.VMEM((8, N_PRE), jnp.float32),
            pltpu.VMEM((5, KPAD), jnp.float32),
            pltpu.VMEM((N_OUT * 4,), jnp.float32),
            pltpu.SMEM((1,), jnp.int32),
            pltpu.VMEM_SHARED((2, 8, N_PRE), jnp.float32),
        ],
    )(_body)
    out = k(tabs, ix)
    return out.reshape(N_IMG, N_OUT, 4)
